# baseline (speedup 1.0000x reference)
import functools

import jax
import jax.numpy as jnp
import numpy as np
from jax import lax
from jax.experimental import pallas as pl
from jax.experimental.pallas import tpu as pltpu
from jax.experimental.pallas import tpu_sc as plsc

BATCH = 4
SEQ = 2048
D = 1024
L = 16
NC, NS = 2, 16
NW = NC * NS
TOK = BATCH * SEQ
ROWS_PW = TOK // NW
C = 16
NCH = ROWS_PW // C
DJ = D // L
_MAGIC = np.int32(0x5F3759DF)


def _ln_body(idx_hbm, tok_hbm, pos_hbm, gam_hbm, bet_hbm, out_hbm,
             idx_v, gam_v, bet_v, tok_buf, pos_buf, res_buf, mv, rv,
             st0, st1, sp0, sp1, so0, so1):
    wid = lax.axis_index("s") * NC + lax.axis_index("c")
    base = wid * ROWS_PW
    pos0 = (wid % (SEQ // ROWS_PW)) * ROWS_PW

    pltpu.sync_copy(idx_hbm.at[pl.ds(base, ROWS_PW)], idx_v)
    pltpu.sync_copy(gam_hbm, gam_v)
    pltpu.sync_copy(bet_hbm, bet_v)

    st = (st0, st1)
    sp = (sp0, sp1)
    so = (so0, so1)

    def start_in(c, b):
        iv = idx_v[pl.ds(c * C, C)]
        pltpu.async_copy(tok_hbm.at[iv], tok_buf.at[b], st[b])
        pltpu.async_copy(pos_hbm.at[pl.ds(pos0 + c * C, C)], pos_buf.at[b],
                         sp[b])

    def wait_in(b):
        pltpu.make_async_copy(tok_hbm.at[pl.ds(0, C)], tok_buf.at[b],
                              st[b]).wait()
        pltpu.make_async_copy(pos_hbm.at[pl.ds(0, C)], pos_buf.at[b],
                              sp[b]).wait()

    def start_out(c, b):
        pltpu.async_copy(res_buf.at[b], out_hbm.at[pl.ds(base + c * C, C)],
                         so[b])

    def wait_out(b):
        pltpu.make_async_copy(res_buf.at[b], out_hbm.at[pl.ds(0, C)],
                              so[b]).wait()

    def compute(b):
        for r in range(C):
            def p1(j, carry):
                a, a2 = carry
                x = (tok_buf[b, r, pl.ds(j * L, L)]
                     + pos_buf[b, r, pl.ds(j * L, L)])
                tok_buf[b, r, pl.ds(j * L, L)] = x
                return a + x, a2 + x * x
            zero = jnp.zeros((L,), jnp.float32)
            a, a2 = lax.fori_loop(0, DJ, p1, (zero, zero))
            mean = jnp.sum(a) * (1.0 / D)
            var = jnp.sum(a2) * (1.0 / D) - mean * mean + 1e-5
            vv = jnp.zeros((L,), jnp.float32) + var
            iv = plsc.bitcast(vv, jnp.int32)
            y = plsc.bitcast(_MAGIC - (iv >> 1), jnp.float32)
            for _ in range(3):
                y = y * (1.5 - 0.5 * vv * y * y)
            mv[r] = mean
            rv[r] = jnp.max(y)

        def p2(j, carry):
            g = gam_v[pl.ds(j * L, L)]
            bt = bet_v[pl.ds(j * L, L)]
            for r in range(C):
                x = tok_buf[b, r, pl.ds(j * L, L)]
                res_buf[b, r, pl.ds(j * L, L)] = (x - mv[r]) * rv[r] * g + bt
            return carry
        lax.fori_loop(0, DJ, p2, 0)

    start_in(0, 0)
    start_in(1, 1)

    def outer(g, carry):
        for b in range(2):
            c = 2 * g + b
            wait_in(b)

            @pl.when(g > 0)
            def _():
                wait_out(b)

            compute(b)
            start_out(c, b)

            @pl.when(g < NCH // 2 - 1)
            def _():
                start_in(c + 2, b)
        return carry

    lax.fori_loop(0, NCH // 2, outer, 0)
    wait_out(0)
    wait_out(1)


@jax.jit
def _run(idx, token_table, pos_table, ln_gamma, ln_beta):
    mesh = plsc.VectorSubcoreMesh(core_axis_name="c", subcore_axis_name="s")
    fn = pl.kernel(
        _ln_body,
        out_type=jax.ShapeDtypeStruct((TOK, D), jnp.float32),
        mesh=mesh,
        compiler_params=pltpu.CompilerParams(needs_layout_passes=False),
        scratch_types=[
            pltpu.VMEM((ROWS_PW,), jnp.int32),
            pltpu.VMEM((D,), jnp.float32),
            pltpu.VMEM((D,), jnp.float32),
            pltpu.VMEM((2, C, D), jnp.float32),
            pltpu.VMEM((2, C, D), jnp.float32),
            pltpu.VMEM((2, C, D), jnp.float32),
            pltpu.SMEM((C,), jnp.float32),
            pltpu.SMEM((C,), jnp.float32),
            pltpu.SemaphoreType.DMA,
            pltpu.SemaphoreType.DMA,
            pltpu.SemaphoreType.DMA,
            pltpu.SemaphoreType.DMA,
            pltpu.SemaphoreType.DMA,
            pltpu.SemaphoreType.DMA,
        ],
    )
    return fn(idx, token_table, pos_table, ln_gamma, ln_beta)


def kernel(input_token, token_table, pos_table, ln_gamma, ln_beta):
    idx = input_token.reshape(-1).astype(jnp.int32)
    out = _run(idx, token_table, pos_table, ln_gamma, ln_beta)
    return out.reshape(BATCH, SEQ, D)


# speedup vs baseline: 1.3070x; 1.3070x over previous
import functools

import jax
import jax.numpy as jnp
import numpy as np
from jax import lax
from jax.experimental import pallas as pl
from jax.experimental.pallas import tpu as pltpu
from jax.experimental.pallas import tpu_sc as plsc

BATCH = 4
SEQ = 2048
D = 1024
L = 16
NC, NS = 2, 16
NW = NC * NS
TOK = BATCH * SEQ
ROWS_PW = TOK // NW
C = 16
NCH = ROWS_PW // C
DJ = D // L
_MAGIC = np.int32(0x5F3759DF)


def _ln_body(idx_hbm, tok_hbm, pos_hbm, gam_hbm, bet_hbm, out_hbm,
             idx_v, gam_v, bet_v, tok_buf, pos_buf, res_buf, mv, rv,
             st0, st1, sp0, sp1, so0, so1):
    wid = lax.axis_index("s") * NC + lax.axis_index("c")
    base = wid * ROWS_PW
    pos0 = (wid % (SEQ // ROWS_PW)) * ROWS_PW

    pltpu.sync_copy(idx_hbm.at[pl.ds(base, ROWS_PW)], idx_v)
    pltpu.sync_copy(gam_hbm, gam_v)
    pltpu.sync_copy(bet_hbm, bet_v)

    st = (st0, st1)
    sp = (sp0, sp1)
    so = (so0, so1)

    def start_in(c, b):
        iv = idx_v[pl.ds(c * C, C)]
        pltpu.async_copy(tok_hbm.at[iv], tok_buf.at[b], st[b])
        pltpu.async_copy(pos_hbm.at[pl.ds(pos0 + c * C, C)], pos_buf.at[b],
                         sp[b])

    def wait_in(b):
        pltpu.make_async_copy(tok_hbm.at[pl.ds(0, C)], tok_buf.at[b],
                              st[b]).wait()
        pltpu.make_async_copy(pos_hbm.at[pl.ds(0, C)], pos_buf.at[b],
                              sp[b]).wait()

    def start_out(c, b):
        pltpu.async_copy(res_buf.at[b], out_hbm.at[pl.ds(base + c * C, C)],
                         so[b])

    def wait_out(b):
        pltpu.make_async_copy(res_buf.at[b], out_hbm.at[pl.ds(0, C)],
                              so[b]).wait()

    def compute(b):
        for r in range(C):
            def p1(j, carry):
                a, a2 = carry
                x = (tok_buf[b, r, pl.ds(j * L, L)]
                     + pos_buf[b, r, pl.ds(j * L, L)])
                tok_buf[b, r, pl.ds(j * L, L)] = x
                return a + x, a2 + x * x
            zero = jnp.zeros((L,), jnp.float32)
            a, a2 = lax.fori_loop(0, DJ, p1, (zero, zero), unroll=8)
            mean = jnp.sum(a) * (1.0 / D)
            var = jnp.sum(a2) * (1.0 / D) - mean * mean + 1e-5
            vv = jnp.zeros((L,), jnp.float32) + var
            iv = plsc.bitcast(vv, jnp.int32)
            y = plsc.bitcast(_MAGIC - (iv >> 1), jnp.float32)
            for _ in range(3):
                y = y * (1.5 - 0.5 * vv * y * y)
            mv[r] = mean
            rv[r] = jnp.max(y)

        def p2(j, carry):
            g = gam_v[pl.ds(j * L, L)]
            bt = bet_v[pl.ds(j * L, L)]
            for r in range(C):
                x = tok_buf[b, r, pl.ds(j * L, L)]
                res_buf[b, r, pl.ds(j * L, L)] = (x - mv[r]) * rv[r] * g + bt
            return carry
        lax.fori_loop(0, DJ, p2, 0)

    start_in(0, 0)
    start_in(1, 1)

    def outer(g, carry):
        for b in range(2):
            c = 2 * g + b
            wait_in(b)

            @pl.when(g > 0)
            def _():
                wait_out(b)

            compute(b)
            start_out(c, b)

            @pl.when(g < NCH // 2 - 1)
            def _():
                start_in(c + 2, b)
        return carry

    lax.fori_loop(0, NCH // 2, outer, 0)
    wait_out(0)
    wait_out(1)


@jax.jit
def _run(idx, token_table, pos_table, ln_gamma, ln_beta):
    mesh = plsc.VectorSubcoreMesh(core_axis_name="c", subcore_axis_name="s")
    fn = pl.kernel(
        _ln_body,
        out_type=jax.ShapeDtypeStruct((TOK, D), jnp.float32),
        mesh=mesh,
        compiler_params=pltpu.CompilerParams(needs_layout_passes=False),
        scratch_types=[
            pltpu.VMEM((ROWS_PW,), jnp.int32),
            pltpu.VMEM((D,), jnp.float32),
            pltpu.VMEM((D,), jnp.float32),
            pltpu.VMEM((2, C, D), jnp.float32),
            pltpu.VMEM((2, C, D), jnp.float32),
            pltpu.VMEM((2, C, D), jnp.float32),
            pltpu.SMEM((C,), jnp.float32),
            pltpu.SMEM((C,), jnp.float32),
            pltpu.SemaphoreType.DMA,
            pltpu.SemaphoreType.DMA,
            pltpu.SemaphoreType.DMA,
            pltpu.SemaphoreType.DMA,
            pltpu.SemaphoreType.DMA,
            pltpu.SemaphoreType.DMA,
        ],
    )
    return fn(idx, token_table, pos_table, ln_gamma, ln_beta)


def kernel(input_token, token_table, pos_table, ln_gamma, ln_beta):
    idx = input_token.reshape(-1).astype(jnp.int32)
    out = _run(idx, token_table, pos_table, ln_gamma, ln_beta)
    return out.reshape(BATCH, SEQ, D)


# speedup vs baseline: 2.2677x; 1.7351x over previous
import functools

import jax
import jax.numpy as jnp
import numpy as np
from jax import lax
from jax.experimental import pallas as pl
from jax.experimental.pallas import tpu as pltpu
from jax.experimental.pallas import tpu_sc as plsc

BATCH = 4
SEQ = 2048
D = 1024
NC, NS = 2, 16
NW = NC * NS
TOK = BATCH * SEQ
ROWS_PW = TOK // NW
GC = 16
NGC = ROWS_PW // GC
NSLOT = 4
OUT_LAG = 2

BLK = 512
EPS = 1e-5


def _gather_body(idx_hbm, tok_hbm, out_hbm, idx_v, buf, sg0, sg1, sg2, sg3,
                 so0, so1, so2, so3):
    wid = lax.axis_index("s") * NC + lax.axis_index("c")
    base = wid * ROWS_PW
    sg = (sg0, sg1, sg2, sg3)
    so = (so0, so1, so2, so3)

    pltpu.sync_copy(idx_hbm.at[pl.ds(base, ROWS_PW)], idx_v)

    def start_g(c, b):
        pltpu.async_copy(tok_hbm.at[idx_v.at[pl.ds(c * GC, GC)]], buf.at[b],
                         sg[b])

    def wait_g(b):
        pltpu.make_async_copy(tok_hbm.at[pl.ds(0, GC)], buf.at[b],
                              sg[b]).wait()

    def start_o(c, b):
        pltpu.async_copy(buf.at[b], out_hbm.at[pl.ds(base + c * GC, GC)],
                         so[b])

    def wait_o(b):
        pltpu.make_async_copy(buf.at[b], out_hbm.at[pl.ds(0, GC)],
                              so[b]).wait()

    for g in range(NGC + OUT_LAG):
        if g < NGC:
            b = g % NSLOT
            if g >= NSLOT:
                wait_o(b)
            start_g(g, b)
        if g >= OUT_LAG:
            c = g - OUT_LAG
            b2 = c % NSLOT
            wait_g(b2)
            start_o(c, b2)
    for c in range(NGC - NSLOT, NGC):
        wait_o(c % NSLOT)


def _sc_gather(idx, token_table):
    mesh = plsc.VectorSubcoreMesh(core_axis_name="c", subcore_axis_name="s")
    fn = pl.kernel(
        _gather_body,
        out_type=jax.ShapeDtypeStruct((TOK, D), jnp.float32),
        mesh=mesh,
        compiler_params=pltpu.CompilerParams(needs_layout_passes=False),
        scratch_types=[
            pltpu.VMEM((ROWS_PW,), jnp.int32),
            pltpu.VMEM((NSLOT, GC, D), jnp.float32),
            pltpu.SemaphoreType.DMA,
            pltpu.SemaphoreType.DMA,
            pltpu.SemaphoreType.DMA,
            pltpu.SemaphoreType.DMA,
            pltpu.SemaphoreType.DMA,
            pltpu.SemaphoreType.DMA,
            pltpu.SemaphoreType.DMA,
            pltpu.SemaphoreType.DMA,
        ],
    )
    return fn(idx, token_table)


def _ln_block(g_ref, p_ref, gam_ref, bet_ref, o_ref):
    x = g_ref[...] + p_ref[...]
    mean = jnp.mean(x, axis=-1, keepdims=True)
    xc = x - mean
    var = jnp.mean(xc * xc, axis=-1, keepdims=True)
    rstd = lax.rsqrt(var + EPS)
    o_ref[...] = xc * rstd * gam_ref[...] + bet_ref[...]


def _tc_ln(gathered, pos_table, ln_gamma, ln_beta):
    nblk = TOK // BLK
    pos_rep = SEQ // BLK
    return pl.pallas_call(
        _ln_block,
        grid=(nblk,),
        in_specs=[
            pl.BlockSpec((BLK, D), lambda i: (i, 0)),
            pl.BlockSpec((BLK, D), lambda i: (i % pos_rep, 0)),
            pl.BlockSpec((1, D), lambda i: (0, 0)),
            pl.BlockSpec((1, D), lambda i: (0, 0)),
        ],
        out_specs=pl.BlockSpec((BLK, D), lambda i: (i, 0)),
        out_shape=jax.ShapeDtypeStruct((TOK, D), jnp.float32),
    )(gathered, pos_table, ln_gamma.reshape(1, D), ln_beta.reshape(1, D))


@jax.jit
def _run(idx, token_table, pos_table, ln_gamma, ln_beta):
    gathered = _sc_gather(idx, token_table)
    return _tc_ln(gathered, pos_table, ln_gamma, ln_beta)


def kernel(input_token, token_table, pos_table, ln_gamma, ln_beta):
    idx = input_token.reshape(-1).astype(jnp.int32)
    out = _run(idx, token_table, pos_table, ln_gamma, ln_beta)
    return out.reshape(BATCH, SEQ, D)


# speedup vs baseline: 2.3074x; 1.0175x over previous
import functools

import jax
import jax.numpy as jnp
import numpy as np
from jax import lax
from jax.experimental import pallas as pl
from jax.experimental.pallas import tpu as pltpu
from jax.experimental.pallas import tpu_sc as plsc

BATCH = 4
SEQ = 2048
D = 1024
NC, NS = 2, 16
NW = NC * NS
TOK = BATCH * SEQ
ROWS_PW = TOK // NW
GC = 16
NGC = ROWS_PW // GC
NSLOT = 4
OUT_LAG = 2

BLK = 1024
EPS = 1e-5


def _gather_body(idx_hbm, tok_hbm, out_hbm, idx_v, buf, sg0, sg1, sg2, sg3,
                 so0, so1, so2, so3):
    wid = lax.axis_index("s") * NC + lax.axis_index("c")
    base = wid * ROWS_PW
    sg = (sg0, sg1, sg2, sg3)
    so = (so0, so1, so2, so3)

    pltpu.sync_copy(idx_hbm.at[pl.ds(base, ROWS_PW)], idx_v)

    def start_g(c, b):
        pltpu.async_copy(tok_hbm.at[idx_v.at[pl.ds(c * GC, GC)]], buf.at[b],
                         sg[b])

    def wait_g(b):
        pltpu.make_async_copy(tok_hbm.at[pl.ds(0, GC)], buf.at[b],
                              sg[b]).wait()

    def start_o(c, b):
        pltpu.async_copy(buf.at[b], out_hbm.at[pl.ds(base + c * GC, GC)],
                         so[b])

    def wait_o(b):
        pltpu.make_async_copy(buf.at[b], out_hbm.at[pl.ds(0, GC)],
                              so[b]).wait()

    for g in range(NGC + OUT_LAG):
        if g < NGC:
            b = g % NSLOT
            if g >= NSLOT:
                wait_o(b)
            start_g(g, b)
        if g >= OUT_LAG:
            c = g - OUT_LAG
            b2 = c % NSLOT
            wait_g(b2)
            start_o(c, b2)
    for c in range(NGC - NSLOT, NGC):
        wait_o(c % NSLOT)


def _sc_gather(idx, token_table):
    mesh = plsc.VectorSubcoreMesh(core_axis_name="c", subcore_axis_name="s")
    fn = pl.kernel(
        _gather_body,
        out_type=jax.ShapeDtypeStruct((TOK, D), jnp.float32),
        mesh=mesh,
        compiler_params=pltpu.CompilerParams(needs_layout_passes=False),
        scratch_types=[
            pltpu.VMEM((ROWS_PW,), jnp.int32),
            pltpu.VMEM((NSLOT, GC, D), jnp.float32),
            pltpu.SemaphoreType.DMA,
            pltpu.SemaphoreType.DMA,
            pltpu.SemaphoreType.DMA,
            pltpu.SemaphoreType.DMA,
            pltpu.SemaphoreType.DMA,
            pltpu.SemaphoreType.DMA,
            pltpu.SemaphoreType.DMA,
            pltpu.SemaphoreType.DMA,
        ],
    )
    return fn(idx, token_table)


def _ln_block(g_ref, p_ref, gam_ref, bet_ref, o_ref):
    x = g_ref[...] + p_ref[...]
    mean = jnp.mean(x, axis=-1, keepdims=True)
    xc = x - mean
    var = jnp.mean(xc * xc, axis=-1, keepdims=True)
    rstd = lax.rsqrt(var + EPS)
    o_ref[...] = xc * rstd * gam_ref[...] + bet_ref[...]


def _tc_ln(gathered, pos_table, ln_gamma, ln_beta):
    nblk = TOK // BLK
    pos_rep = SEQ // BLK
    return pl.pallas_call(
        _ln_block,
        grid=(nblk,),
        in_specs=[
            pl.BlockSpec((BLK, D), lambda i: (i, 0)),
            pl.BlockSpec((BLK, D), lambda i: (i % pos_rep, 0)),
            pl.BlockSpec((1, D), lambda i: (0, 0)),
            pl.BlockSpec((1, D), lambda i: (0, 0)),
        ],
        out_specs=pl.BlockSpec((BLK, D), lambda i: (i, 0)),
        out_shape=jax.ShapeDtypeStruct((TOK, D), jnp.float32),
    )(gathered, pos_table, ln_gamma.reshape(1, D), ln_beta.reshape(1, D))


@jax.jit
def _run(idx, token_table, pos_table, ln_gamma, ln_beta):
    gathered = _sc_gather(idx, token_table)
    return _tc_ln(gathered, pos_table, ln_gamma, ln_beta)


def kernel(input_token, token_table, pos_table, ln_gamma, ln_beta):
    idx = input_token.reshape(-1).astype(jnp.int32)
    out = _run(idx, token_table, pos_table, ln_gamma, ln_beta)
    return out.reshape(BATCH, SEQ, D)
